# baseline (speedup 1.0000x reference)
import functools

import jax
import jax.numpy as jnp
from jax import lax
from jax.experimental import pallas as pl
from jax.experimental.pallas import tpu as pltpu
from jax.experimental.pallas import tpu_sc as plsc

NC = 2
NS = 16
L = 16
NW = NC * NS
B = 8960
WINW = 384
WPITCH = 385


def _spmv_grid(n, nnz):
    nslice = n // NS
    j_full = nnz // B
    nvec = B // L
    mesh = plsc.VectorSubcoreMesh(core_axis_name="c", subcore_axis_name="s")

    @functools.partial(
        pl.kernel,
        out_type=jax.ShapeDtypeStruct((NC, n), jnp.float32),
        mesh=mesh,
        compiler_params=pltpu.CompilerParams(needs_layout_passes=False),
        scratch_types=[
            pltpu.VMEM((n,), jnp.float32),
            pltpu.VMEM((B,), jnp.float32),
            pltpu.VMEM((B,), jnp.float32),
            pltpu.VMEM((B,), jnp.int32),
            pltpu.VMEM((B,), jnp.int32),
            pltpu.VMEM((B,), jnp.int32),
            pltpu.VMEM((B,), jnp.int32),
            pltpu.VMEM((L * WPITCH,), jnp.float32),
            pltpu.VMEM((WINW,), jnp.float32),
            pltpu.VMEM((WINW,), jnp.int32),
            pltpu.VMEM((WINW,), jnp.int32),
            pltpu.VMEM_SHARED((n,), jnp.float32),
            pltpu.SemaphoreType.DMA((2,)),
        ],
    )
    def k(vals_hbm, rows_hbm, cols_hbm, x_hbm, tval, trow, tcol, out,
          x_v, vals0, vals1, cols0, cols1, rows0, rows1,
          win, yrow, wiota, widx, y_sh, dsem):
        cid = lax.axis_index("c")
        sid = lax.axis_index("s")
        wid = sid * NC + cid
        vals_v = (vals0, vals1)
        cols_v = (cols0, cols1)
        rows_v = (rows0, rows1)
        zero16 = jnp.zeros((L,), jnp.float32)

        def issue_in(b, r):
            base = (r * NW + wid) * B
            pltpu.async_copy(vals_hbm.at[pl.ds(base, B)], vals_v[b],
                             dsem.at[b])
            pltpu.async_copy(cols_hbm.at[pl.ds(base, B)], cols_v[b],
                             dsem.at[b])
            pltpu.async_copy(rows_hbm.at[pl.ds(base, B)], rows_v[b],
                             dsem.at[b])

        def wait_in(b, r):
            base = (r * NW + wid) * B
            pltpu.make_async_copy(vals_hbm.at[pl.ds(base, B)], vals_v[b],
                                  dsem.at[b]).wait()
            pltpu.make_async_copy(cols_hbm.at[pl.ds(base, B)], cols_v[b],
                                  dsem.at[b]).wait()
            pltpu.make_async_copy(rows_hbm.at[pl.ds(base, B)], rows_v[b],
                                  dsem.at[b]).wait()

        def process(vals_b, cols_b, rows_b):
            lo = jnp.min(rows_b[pl.ds(0, L)])
            hi = jnp.max(rows_b[pl.ds((nvec - 1) * L, L)])
            lane_base = lax.iota(jnp.int32, L) * WPITCH

            @pl.when(hi - lo < WINW)
            def _():
                @plsc.parallel_loop(0, nvec, unroll=8)
                def gm(i):
                    c16 = cols_b[pl.ds(i * L, L)]
                    v16 = vals_b[pl.ds(i * L, L)]
                    r16 = rows_b[pl.ds(i * L, L)]
                    xv = plsc.load_gather(x_v, [c16])
                    plsc.addupdate_scatter(
                        win, [lane_base + (r16 - lo)], xv * v16)

                nred = (hi - lo + L) // L

                def red(jv, _):
                    jb = jv * L
                    acc = zero16
                    for l in range(L):
                        acc = acc + win[pl.ds(l * WPITCH + jb, L)]
                    for l in range(L):
                        win[pl.ds(l * WPITCH + jb, L)] = zero16
                    yrow[pl.ds(jb, L)] = acc
                    widx[pl.ds(jb, L)] = jnp.minimum(
                        wiota[pl.ds(jb, L)] + lo, n - 1)
                    return 0
                lax.fori_loop(0, nred, red, 0)

                pltpu.sync_copy(yrow, y_sh.at[widx], add=True)

                def zr(jv, _):
                    yrow[pl.ds(jv * L, L)] = zero16
                    return 0
                lax.fori_loop(0, nred, zr, 0)

            @pl.when(hi - lo >= WINW)
            def _():
                def gm(i, _):
                    c16 = cols_b[pl.ds(i * L, L)]
                    v16 = vals_b[pl.ds(i * L, L)]
                    xv = plsc.load_gather(x_v, [c16])
                    vals_b[pl.ds(i * L, L)] = xv * v16
                    return 0
                lax.fori_loop(0, nvec, gm, 0)
                pltpu.sync_copy(vals_b, y_sh.at[rows_b], add=True)

        def bi(i, _):
            iv = lax.iota(jnp.int32, L) + jnp.int32(i * L)
            wiota[pl.ds(i * L, L)] = iv
            widx[pl.ds(i * L, L)] = iv
            yrow[pl.ds(i * L, L)] = zero16
            return 0
        lax.fori_loop(0, WINW // L, bi, 0)

        def zw(i, _):
            win[pl.ds(i * L, L)] = zero16
            return 0
        lax.fori_loop(0, (L * WPITCH) // L, zw, 0)

        def z(i, _):
            vals0[pl.ds(i * L, L)] = zero16
            return 0
        lax.fori_loop(0, nslice // L, z, 0)
        pltpu.sync_copy(vals0.at[pl.ds(0, nslice)],
                        y_sh.at[pl.ds(sid * nslice, nslice)])
        pltpu.sync_copy(x_hbm, x_v)
        plsc.subcore_barrier()

        nsub = (j_full + NW - 1 - wid) // NW

        @pl.when(nsub > 0)
        def _():
            issue_in(0, 0)

        @pl.loop(0, nsub, step=2)
        def _(outer):
            for b in range(2):
                r = outer + b

                @pl.when(r < nsub)
                def _(r=r, b=b):
                    wait_in(b, r)

                    @pl.when(r + 1 < nsub)
                    def _(r=r, b=b):
                        issue_in(1 - b, r + 1)
                    process(vals_v[b], cols_v[b], rows_v[b])

        @pl.when(wid == NW - 1)
        def _():
            pltpu.sync_copy(tval, vals0)
            pltpu.sync_copy(tcol, cols0)
            pltpu.sync_copy(trow, rows0)
            process(vals0, cols0, rows0)

        plsc.subcore_barrier()
        pltpu.sync_copy(y_sh.at[pl.ds(sid * nslice, nslice)],
                        out.at[cid, pl.ds(sid * nslice, nslice)])

    return k


def _combine_body(p_ref, o_ref):
    o_ref[...] = p_ref[0, :] + p_ref[1, :]


def kernel(H_vals, H_rows, H_cols, x):
    n = x.shape[0]
    nnz = H_vals.shape[0]
    j_full = nnz // B
    tail = nnz - j_full * B
    rows = H_rows.astype(jnp.int32)
    cols = H_cols.astype(jnp.int32)
    tval = jnp.zeros((B,), jnp.float32).at[:tail].set(H_vals[j_full * B:])
    trow = jnp.full((B,), rows[-1], jnp.int32).at[:tail].set(rows[j_full * B:])
    tcol = jnp.zeros((B,), jnp.int32).at[:tail].set(cols[j_full * B:])
    partial = _spmv_grid(n, nnz)(H_vals, rows, cols, x, tval, trow, tcol)
    y = pl.pallas_call(
        _combine_body,
        out_shape=jax.ShapeDtypeStruct((n,), jnp.float32),
    )(partial)
    return y


# speedup vs baseline: 1.0208x; 1.0208x over previous
import functools

import jax
import jax.numpy as jnp
from jax import lax
from jax.experimental import pallas as pl
from jax.experimental.pallas import tpu as pltpu
from jax.experimental.pallas import tpu_sc as plsc

NC = 2
NS = 16
L = 16
NW = NC * NS
B = 8960
WINW = 384
WPITCH = 385


def _spmv_grid(n, nnz):
    nslice = n // NS
    j_full = nnz // B
    nvec = B // L
    mesh = plsc.VectorSubcoreMesh(core_axis_name="c", subcore_axis_name="s")

    @functools.partial(
        pl.kernel,
        out_type=jax.ShapeDtypeStruct((NC, n), jnp.float32),
        mesh=mesh,
        compiler_params=pltpu.CompilerParams(needs_layout_passes=False),
        scratch_types=[
            pltpu.VMEM((n,), jnp.float32),
            pltpu.VMEM((B,), jnp.float32),
            pltpu.VMEM((B,), jnp.float32),
            pltpu.VMEM((B,), jnp.int32),
            pltpu.VMEM((B,), jnp.int32),
            pltpu.VMEM((B,), jnp.int32),
            pltpu.VMEM((B,), jnp.int32),
            pltpu.VMEM((L * WPITCH,), jnp.float32),
            pltpu.VMEM((WINW,), jnp.float32),
            pltpu.VMEM((WINW,), jnp.int32),
            pltpu.VMEM((WINW,), jnp.int32),
            pltpu.VMEM_SHARED((n,), jnp.float32),
            pltpu.SemaphoreType.DMA((2,)),
        ],
    )
    def k(vals_hbm, rows_hbm, cols_hbm, x_hbm, tval, trow, tcol, out,
          x_v, vals0, vals1, cols0, cols1, rows0, rows1,
          win, yrow, wiota, widx, y_sh, dsem):
        cid = lax.axis_index("c")
        sid = lax.axis_index("s")
        wid = sid * NC + cid
        vals_v = (vals0, vals1)
        cols_v = (cols0, cols1)
        rows_v = (rows0, rows1)
        zero16 = jnp.zeros((L,), jnp.float32)

        def issue_in(b, r):
            base = (r * NW + wid) * B
            pltpu.async_copy(vals_hbm.at[pl.ds(base, B)], vals_v[b],
                             dsem.at[b])
            pltpu.async_copy(cols_hbm.at[pl.ds(base, B)], cols_v[b],
                             dsem.at[b])
            pltpu.async_copy(rows_hbm.at[pl.ds(base, B)], rows_v[b],
                             dsem.at[b])

        def wait_in(b, r):
            base = (r * NW + wid) * B
            pltpu.make_async_copy(vals_hbm.at[pl.ds(base, B)], vals_v[b],
                                  dsem.at[b]).wait()
            pltpu.make_async_copy(cols_hbm.at[pl.ds(base, B)], cols_v[b],
                                  dsem.at[b]).wait()
            pltpu.make_async_copy(rows_hbm.at[pl.ds(base, B)], rows_v[b],
                                  dsem.at[b]).wait()

        def process(vals_b, cols_b, rows_b):
            lo = jnp.min(rows_b[pl.ds(0, L)])
            hi = jnp.max(rows_b[pl.ds((nvec - 1) * L, L)])
            lane_base = lax.iota(jnp.int32, L) * WPITCH

            @pl.when(hi - lo < WINW)
            def _():
                @plsc.parallel_loop(0, nvec, unroll=4)
                def gm(i):
                    c16 = cols_b[pl.ds(i * L, L)]
                    v16 = vals_b[pl.ds(i * L, L)]
                    r16 = rows_b[pl.ds(i * L, L)]
                    xv = plsc.load_gather(x_v, [c16])
                    plsc.addupdate_scatter(
                        win, [lane_base + (r16 - lo)], xv * v16)

                nred = (hi - lo + L) // L

                def red(jv, _):
                    jb = jv * L
                    acc = zero16
                    for l in range(L):
                        acc = acc + win[pl.ds(l * WPITCH + jb, L)]
                    for l in range(L):
                        win[pl.ds(l * WPITCH + jb, L)] = zero16
                    yrow[pl.ds(jb, L)] = acc
                    widx[pl.ds(jb, L)] = jnp.minimum(
                        wiota[pl.ds(jb, L)] + lo, n - 1)
                    return 0
                lax.fori_loop(0, nred, red, 0)

                pltpu.sync_copy(yrow, y_sh.at[widx], add=True)

                def zr(jv, _):
                    yrow[pl.ds(jv * L, L)] = zero16
                    return 0
                lax.fori_loop(0, nred, zr, 0)

            @pl.when(hi - lo >= WINW)
            def _():
                def gm(i, _):
                    c16 = cols_b[pl.ds(i * L, L)]
                    v16 = vals_b[pl.ds(i * L, L)]
                    xv = plsc.load_gather(x_v, [c16])
                    vals_b[pl.ds(i * L, L)] = xv * v16
                    return 0
                lax.fori_loop(0, nvec, gm, 0)
                pltpu.sync_copy(vals_b, y_sh.at[rows_b], add=True)

        def bi(i, _):
            iv = lax.iota(jnp.int32, L) + jnp.int32(i * L)
            wiota[pl.ds(i * L, L)] = iv
            widx[pl.ds(i * L, L)] = iv
            yrow[pl.ds(i * L, L)] = zero16
            return 0
        lax.fori_loop(0, WINW // L, bi, 0)

        def zw(i, _):
            win[pl.ds(i * L, L)] = zero16
            return 0
        lax.fori_loop(0, (L * WPITCH) // L, zw, 0)

        def z(i, _):
            vals0[pl.ds(i * L, L)] = zero16
            return 0
        lax.fori_loop(0, nslice // L, z, 0)
        pltpu.sync_copy(vals0.at[pl.ds(0, nslice)],
                        y_sh.at[pl.ds(sid * nslice, nslice)])
        pltpu.sync_copy(x_hbm, x_v)
        plsc.subcore_barrier()

        nsub = (j_full + NW - 1 - wid) // NW

        @pl.when(nsub > 0)
        def _():
            issue_in(0, 0)

        @pl.loop(0, nsub, step=2)
        def _(outer):
            for b in range(2):
                r = outer + b

                @pl.when(r < nsub)
                def _(r=r, b=b):
                    wait_in(b, r)

                    @pl.when(r + 1 < nsub)
                    def _(r=r, b=b):
                        issue_in(1 - b, r + 1)
                    process(vals_v[b], cols_v[b], rows_v[b])

        @pl.when(wid == NW - 1)
        def _():
            pltpu.sync_copy(tval, vals0)
            pltpu.sync_copy(tcol, cols0)
            pltpu.sync_copy(trow, rows0)
            process(vals0, cols0, rows0)

        plsc.subcore_barrier()
        pltpu.sync_copy(y_sh.at[pl.ds(sid * nslice, nslice)],
                        out.at[cid, pl.ds(sid * nslice, nslice)])

    return k


def _combine_body(p_ref, o_ref):
    o_ref[...] = p_ref[0, :] + p_ref[1, :]


def kernel(H_vals, H_rows, H_cols, x):
    n = x.shape[0]
    nnz = H_vals.shape[0]
    j_full = nnz // B
    tail = nnz - j_full * B
    rows = H_rows.astype(jnp.int32)
    cols = H_cols.astype(jnp.int32)
    tval = jnp.zeros((B,), jnp.float32).at[:tail].set(H_vals[j_full * B:])
    trow = jnp.full((B,), rows[-1], jnp.int32).at[:tail].set(rows[j_full * B:])
    tcol = jnp.zeros((B,), jnp.int32).at[:tail].set(cols[j_full * B:])
    partial = _spmv_grid(n, nnz)(H_vals, rows, cols, x, tval, trow, tcol)
    y = pl.pallas_call(
        _combine_body,
        out_shape=jax.ShapeDtypeStruct((n,), jnp.float32),
    )(partial)
    return y
